# baseline (speedup 1.0000x reference)
import functools

import jax
import jax.numpy as jnp
from jax import lax
from jax.experimental import pallas as pl
from jax.experimental.pallas import tpu as pltpu
from jax.experimental.pallas import tpu_sc as plsc

N = 10000
E = 320000
D = 128
L = 3
C = 10

NC = 2
NS = 16
NW = NC * NS
G = 80
CHT = 256
CH0 = 192
CH1 = CHT - CH0
EP = NS * CHT * G
NPAD = 10240
RPT = NPAD // NS
DUMMY = N
BR = 1000
NB = N // BR

_mesh = plsc.VectorSubcoreMesh(core_axis_name="c", subcore_axis_name="s")


@functools.partial(
    pl.kernel,
    out_type=jax.ShapeDtypeStruct((3 * NC * NPAD,), jnp.float32),
    mesh=_mesh,
    scratch_types=[
        pltpu.VMEM_SHARED((NPAD,), jnp.float32),
        pltpu.VMEM((G,), jnp.float32),
        pltpu.VMEM((CHT // 2, G), jnp.int32),
        pltpu.VMEM((RPT,), jnp.float32),
    ],
    compiler_params=pltpu.CompilerParams(use_tc_tiling_on_sc=False),
)
def _sc_degree(dst_hbm, zeros1_hbm, ones_hbm, out_hbm, deg_acc, ones_v, didx,
               zb):
    c = lax.axis_index("c")
    s = lax.axis_index("s")
    pltpu.sync_copy(ones_hbm, ones_v)
    pltpu.sync_copy(zeros1_hbm, zb)
    for b in range(3):
        pltpu.sync_copy(zb, deg_acc.at[pl.ds(s * RPT, RPT)])
        plsc.subcore_barrier()
        base = (b * NS + s) * CHT + c * (CHT // 2)
        pltpu.sync_copy(dst_hbm.at[pl.ds(base, CHT // 2)], didx)

        def chunk(j, _):
            pltpu.sync_copy(ones_v, deg_acc.at[didx.at[j]], add=True)
            return _

        lax.fori_loop(0, CHT // 2, chunk, None)
        plsc.subcore_barrier()
        off = (b * NC + c) * NPAD + s * RPT
        pltpu.sync_copy(deg_acc.at[pl.ds(s * RPT, RPT)], zb)
        pltpu.sync_copy(zb, out_hbm.at[pl.ds(off, RPT)])
        pltpu.sync_copy(zeros1_hbm, zb)
        plsc.subcore_barrier()


@functools.partial(
    pl.kernel,
    out_type=jax.ShapeDtypeStruct((3 * NC * NPAD, D), jnp.float32),
    mesh=_mesh,
    scratch_types=[
        pltpu.VMEM_SHARED((NPAD, D), jnp.float32),
        pltpu.VMEM((64, D), jnp.float32),
        pltpu.VMEM((G, D // 2), jnp.int32),
        pltpu.VMEM((G, D // 2), jnp.int32),
        pltpu.VMEM((G, D), jnp.float32),
        pltpu.VMEM((G, D), jnp.float32),
        pltpu.VMEM((2 * G,), jnp.int32),
        pltpu.VMEM((2 * G,), jnp.int32),
        pltpu.VMEM((2 * G,), jnp.int32),
        pltpu.VMEM((2 * G,), jnp.int32),
        pltpu.SemaphoreType.DMA,
        pltpu.SemaphoreType.DMA,
        pltpu.SemaphoreType.DMA,
        pltpu.SemaphoreType.DMA,
        pltpu.SemaphoreType.DMA,
        pltpu.SemaphoreType.DMA,
        pltpu.SemaphoreType.DMA,
        pltpu.SemaphoreType.DMA,
    ],
    compiler_params=pltpu.CompilerParams(use_tc_tiling_on_sc=False,
                                         needs_layout_passes=False),
)
def _sc_scatter(table_hbm, comb_hbm, zrows_hbm, out_hbm,
                acc, zdbuf, gi0, gi1, g0, g1, xa0, xa1, xb0, xb1,
                semg0, semg1, sems0, sems1, semx0, semx1, semx2, semx3):
    c = lax.axis_index("c")
    s = lax.axis_index("s")
    nzc = RPT // 64
    coff = jnp.where(c == 0, 0, CH0)
    nch = jnp.where(c == 0, CH0, CH1)
    nquad = nch // 4
    himask = jnp.int32(-65536)

    def src(x):
        return x.at[pl.ds(0, G)]

    def dst(x):
        return x.at[pl.ds(G, G)]

    def unpack(gi, gf):
        def row(r, _):
            for k in range(4):
                v = gi[r, pl.ds(16 * k, 16)]
                gf[r, pl.ds(32 * k, 16)] = plsc.bitcast(
                    lax.shift_left(v, 16), jnp.float32)
                gf[r, pl.ds(32 * k + 16, 16)] = plsc.bitcast(
                    lax.bitwise_and(v, himask), jnp.float32)
            return _

        lax.fori_loop(0, G, row, None)

    for b in range(3):
        pltpu.sync_copy(zrows_hbm, zdbuf)

        def zero(h, _):
            pltpu.sync_copy(zdbuf, acc.at[pl.ds(s * RPT + h * 64, 64)])
            return _

        lax.fori_loop(0, nzc, zero, None)
        plsc.subcore_barrier()
        base = (b * NS + s) * CHT + coff
        pltpu.sync_copy(comb_hbm.at[base], xa0)
        pltpu.sync_copy(comb_hbm.at[base + 1], xa1)
        pltpu.async_copy(table_hbm.at[src(xa0)], gi0, semg0)
        pltpu.async_copy(table_hbm.at[src(xa1)], gi1, semg1)
        pltpu.async_copy(comb_hbm.at[base + 2], xb0, semx2)
        pltpu.async_copy(comb_hbm.at[base + 3], xb1, semx3)

        def quad(i, _):
            q = 4 * i
            pltpu.make_async_copy(table_hbm.at[src(xa0)], gi0, semg0).wait()
            unpack(gi0, g0)
            pltpu.async_copy(g0, acc.at[dst(xa0)], sems0, add=True)
            pltpu.make_async_copy(comb_hbm.at[base + q + 2], xb0, semx2).wait()
            pltpu.async_copy(table_hbm.at[src(xb0)], gi0, semg0)
            pltpu.make_async_copy(table_hbm.at[src(xa1)], gi1, semg1).wait()
            unpack(gi1, g1)
            pltpu.async_copy(g1, acc.at[dst(xa1)], sems1, add=True)
            pltpu.make_async_copy(comb_hbm.at[base + q + 3], xb1, semx3).wait()
            pltpu.async_copy(table_hbm.at[src(xb1)], gi1, semg1)
            pltpu.make_async_copy(g0, acc.at[dst(xa0)], sems0).wait()

            @pl.when(q + 4 < nch)
            def _():
                pltpu.async_copy(comb_hbm.at[base + q + 4], xa0, semx0)

            pltpu.make_async_copy(g1, acc.at[dst(xa1)], sems1).wait()

            @pl.when(q + 5 < nch)
            def _():
                pltpu.async_copy(comb_hbm.at[base + q + 5], xa1, semx1)

            pltpu.make_async_copy(table_hbm.at[src(xb0)], gi0, semg0).wait()
            unpack(gi0, g0)
            pltpu.async_copy(g0, acc.at[dst(xb0)], sems0, add=True)

            @pl.when(q + 4 < nch)
            def _():
                pltpu.make_async_copy(comb_hbm.at[base + q + 4], xa0,
                                      semx0).wait()
                pltpu.async_copy(table_hbm.at[src(xa0)], gi0, semg0)

            pltpu.make_async_copy(table_hbm.at[src(xb1)], gi1, semg1).wait()
            unpack(gi1, g1)
            pltpu.async_copy(g1, acc.at[dst(xb1)], sems1, add=True)

            @pl.when(q + 5 < nch)
            def _():
                pltpu.make_async_copy(comb_hbm.at[base + q + 5], xa1,
                                      semx1).wait()
                pltpu.async_copy(table_hbm.at[src(xa1)], gi1, semg1)

            pltpu.make_async_copy(g0, acc.at[dst(xb0)], sems0).wait()

            @pl.when(q + 6 < nch)
            def _():
                pltpu.async_copy(comb_hbm.at[base + q + 6], xb0, semx2)

            pltpu.make_async_copy(g1, acc.at[dst(xb1)], sems1).wait()

            @pl.when(q + 7 < nch)
            def _():
                pltpu.async_copy(comb_hbm.at[base + q + 7], xb1, semx3)

            return _

        lax.fori_loop(0, nquad, quad, None)
        plsc.subcore_barrier()
        off = (b * NC + c) * NPAD + s * RPT

        def dump(h, _):
            pltpu.sync_copy(acc.at[pl.ds(s * RPT + h * 64, 64)], zdbuf)
            pltpu.sync_copy(zdbuf, out_hbm.at[pl.ds(off + h * 64, 64)])
            return _

        lax.fori_loop(0, nzc, dump, None)
        plsc.subcore_barrier()


def _t0_body(x_ref, w_ref, degp_ref, scaled_ref, dinv_ref):
    deg = 1.0 + degp_ref[0, 0] + degp_ref[0, 1]
    dv = lax.rsqrt(deg)
    dinv_ref[0] = dv
    scaled_ref[0] = jnp.dot(x_ref[0], w_ref[0],
                            preferred_element_type=jnp.float32) * dv


def _t0(x, w0, degp):
    return pl.pallas_call(
        _t0_body,
        grid=(3, NB),
        in_specs=[
            pl.BlockSpec((1, BR, D), lambda b, i: (b, i, 0)),
            pl.BlockSpec((1, D, D), lambda b, i: (b, 0, 0)),
            pl.BlockSpec((1, 2, BR, 1), lambda b, i: (b, 0, i, 0)),
        ],
        out_specs=[
            pl.BlockSpec((1, BR, D), lambda b, i: (b, i, 0)),
            pl.BlockSpec((1, BR, 1), lambda b, i: (b, i, 0)),
        ],
        out_shape=[
            jax.ShapeDtypeStruct((3, N, D), jnp.float32),
            jax.ShapeDtypeStruct((3, N, 1), jnp.float32),
        ],
    )(x, w0, degp)


def _post_layer(x, sc, p0, p1, dv, bl, gl, bel):
    pre = (p0 + p1 + sc) * dv + bl[None, :]
    mu = jnp.mean(pre, axis=-1, keepdims=True)
    var = jnp.mean((pre - mu) ** 2, axis=-1, keepdims=True)
    h = (pre - mu) * lax.rsqrt(var + 1e-5) * gl[None, :] + bel[None, :]
    return x + jnp.maximum(h, 0.0)


def _tmid_body(x_ref, s_ref, p_ref, dinv_ref, b_ref, g_ref, be_ref, wn_ref,
               xn_ref, sn_ref):
    dv = dinv_ref[0]
    xn = _post_layer(x_ref[0], s_ref[0], p_ref[0, 0], p_ref[0, 1], dv,
                     b_ref[0, 0], g_ref[0, 0], be_ref[0, 0])
    xn_ref[0] = xn
    sn_ref[0] = jnp.dot(xn, wn_ref[0], preferred_element_type=jnp.float32) * dv


def _tmid(x, scaled, p, dinv, bl, gl, bel, wn):
    return pl.pallas_call(
        _tmid_body,
        grid=(3, NB),
        in_specs=[
            pl.BlockSpec((1, BR, D), lambda b, i: (b, i, 0)),
            pl.BlockSpec((1, BR, D), lambda b, i: (b, i, 0)),
            pl.BlockSpec((1, 2, BR, D), lambda b, i: (b, 0, i, 0)),
            pl.BlockSpec((1, BR, 1), lambda b, i: (b, i, 0)),
            pl.BlockSpec((1, 1, D), lambda b, i: (b, 0, 0)),
            pl.BlockSpec((1, 1, D), lambda b, i: (b, 0, 0)),
            pl.BlockSpec((1, 1, D), lambda b, i: (b, 0, 0)),
            pl.BlockSpec((1, D, D), lambda b, i: (b, 0, 0)),
        ],
        out_specs=[
            pl.BlockSpec((1, BR, D), lambda b, i: (b, i, 0)),
            pl.BlockSpec((1, BR, D), lambda b, i: (b, i, 0)),
        ],
        out_shape=[
            jax.ShapeDtypeStruct((3, N, D), jnp.float32),
            jax.ShapeDtypeStruct((3, N, D), jnp.float32),
        ],
    )(x, scaled, p, dinv, bl, gl, bel, wn)


def _tfin_body(x_ref, s_ref, p_ref, dinv_ref, b_ref, g_ref, be_ref, cw_ref,
               cb_ref, out_ref):
    b = pl.program_id(1)
    dv = dinv_ref[0]
    xn = _post_layer(x_ref[0], s_ref[0], p_ref[0, 0], p_ref[0, 1], dv,
                     b_ref[0, 0], g_ref[0, 0], be_ref[0, 0])
    contrib = jnp.dot(xn, cw_ref[0], preferred_element_type=jnp.float32)

    @pl.when(b == 0)
    def _():
        out_ref[...] = contrib + cb_ref[...]

    @pl.when(b > 0)
    def _():
        out_ref[...] += contrib


def _tfin(x, scaled, p, dinv, bl, gl, bel, cw, cb):
    return pl.pallas_call(
        _tfin_body,
        grid=(NB, 3),
        in_specs=[
            pl.BlockSpec((1, BR, D), lambda i, b: (b, i, 0)),
            pl.BlockSpec((1, BR, D), lambda i, b: (b, i, 0)),
            pl.BlockSpec((1, 2, BR, D), lambda i, b: (b, 0, i, 0)),
            pl.BlockSpec((1, BR, 1), lambda i, b: (b, i, 0)),
            pl.BlockSpec((1, 1, D), lambda i, b: (b, 0, 0)),
            pl.BlockSpec((1, 1, D), lambda i, b: (b, 0, 0)),
            pl.BlockSpec((1, 1, D), lambda i, b: (b, 0, 0)),
            pl.BlockSpec((1, D, C), lambda i, b: (b, 0, 0)),
            pl.BlockSpec((1, C), lambda i, b: (0, 0)),
        ],
        out_specs=pl.BlockSpec((BR, C), lambda i, b: (i, 0)),
        out_shape=jax.ShapeDtypeStruct((N, C), jnp.float32),
    )(x, scaled, p, dinv, bl, gl, bel, cw, cb)


def kernel(x_renormalized, edge_index_renormalized, x_vanilla,
           edge_index_vanilla, x_third, edge_index_third,
           W_ren, b_ren, g_ren, be_ren, W_van, b_van, g_van, be_van,
           W_thd, b_thd, g_thd, be_thd, clf_W, clf_b):
    x = jnp.stack([x_renormalized, x_vanilla, x_third])
    wm = jnp.stack([W_ren, W_van, W_thd])
    bm = jnp.stack([b_ren, b_van, b_thd])
    gm = jnp.stack([g_ren, g_van, g_thd])
    bem = jnp.stack([be_ren, be_van, be_thd])

    srcs = jnp.stack([edge_index_renormalized[0], edge_index_vanilla[0],
                      edge_index_third[0]]).astype(jnp.int32)
    dsts = jnp.stack([edge_index_renormalized[1], edge_index_vanilla[1],
                      edge_index_third[1]]).astype(jnp.int32)
    offs = (jnp.arange(3, dtype=jnp.int32) * N)[:, None]
    pad = EP - E
    src_p = jnp.concatenate(
        [srcs + offs, jnp.broadcast_to(offs, (3, pad))], axis=1)
    dst_p = jnp.concatenate(
        [dsts, jnp.full((3, pad), DUMMY, jnp.int32)], axis=1)
    src_hbm = src_p.reshape(3 * NS * CHT, G)
    dst_hbm = dst_p.reshape(3 * NS * CHT, G)
    comb_hbm = jnp.concatenate([src_hbm, dst_hbm], axis=1)

    zeros1 = jnp.zeros((RPT,), jnp.float32)
    ones_g = jnp.ones((G,), jnp.float32)
    zrows = jnp.zeros((64, D), jnp.float32)

    degp = _sc_degree(dst_hbm, zeros1, ones_g).reshape(3, NC, NPAD, 1)
    scaled, dinv = _t0(x, wm[:, 0], degp)

    for l in range(L):
        perm = scaled.reshape(3 * N, 4, 2, 16).swapaxes(2, 3)
        table = jax.lax.bitcast_convert_type(
            perm.astype(jnp.bfloat16).reshape(3 * N, D // 2, 2), jnp.int32)
        p = _sc_scatter(table, comb_hbm, zrows).reshape(3, NC, NPAD, D)
        if l < L - 1:
            x, scaled = _tmid(x, scaled, p, dinv, bm[:, l:l + 1],
                              gm[:, l:l + 1], bem[:, l:l + 1], wm[:, l + 1])
        else:
            out = _tfin(x, scaled, p, dinv, bm[:, l:l + 1], gm[:, l:l + 1],
                        bem[:, l:l + 1], clf_W.reshape(3, D, C),
                        clf_b.reshape(1, C))
    return out


# speedup vs baseline: 1.2681x; 1.2681x over previous
import functools

import jax
import jax.numpy as jnp
from jax import lax
from jax.experimental import pallas as pl
from jax.experimental.pallas import tpu as pltpu
from jax.experimental.pallas import tpu_sc as plsc

N = 10000
E = 320000
D = 128
L = 3
C = 10

NC = 2
NS = 16
NW = NC * NS
G = 80
CHT = 256
CH0 = 148
CH1 = CHT - CH0
EP = NS * CHT * G
NPAD = 10240
RPT = NPAD // NS
DUMMY = N
BR = 1000
NB = N // BR

_mesh = plsc.VectorSubcoreMesh(core_axis_name="c", subcore_axis_name="s")


@functools.partial(
    pl.kernel,
    out_type=jax.ShapeDtypeStruct((3 * NC * NPAD,), jnp.float32),
    mesh=_mesh,
    scratch_types=[
        pltpu.VMEM_SHARED((NPAD,), jnp.float32),
        pltpu.VMEM((G,), jnp.float32),
        pltpu.VMEM((CHT // 2, G), jnp.int32),
        pltpu.VMEM((RPT,), jnp.float32),
    ],
    compiler_params=pltpu.CompilerParams(use_tc_tiling_on_sc=False),
)
def _sc_degree(dst_hbm, zeros1_hbm, ones_hbm, out_hbm, deg_acc, ones_v, didx,
               zb):
    c = lax.axis_index("c")
    s = lax.axis_index("s")
    pltpu.sync_copy(ones_hbm, ones_v)
    pltpu.sync_copy(zeros1_hbm, zb)
    for b in range(3):
        pltpu.sync_copy(zb, deg_acc.at[pl.ds(s * RPT, RPT)])
        plsc.subcore_barrier()
        base = (b * NS + s) * CHT + c * (CHT // 2)
        pltpu.sync_copy(dst_hbm.at[pl.ds(base, CHT // 2)], didx)

        def chunk(j, _):
            pltpu.sync_copy(ones_v, deg_acc.at[didx.at[j]], add=True)
            return _

        lax.fori_loop(0, CHT // 2, chunk, None)
        plsc.subcore_barrier()
        off = (b * NC + c) * NPAD + s * RPT
        pltpu.sync_copy(deg_acc.at[pl.ds(s * RPT, RPT)], zb)
        pltpu.sync_copy(zb, out_hbm.at[pl.ds(off, RPT)])
        pltpu.sync_copy(zeros1_hbm, zb)
        plsc.subcore_barrier()


@functools.partial(
    pl.kernel,
    out_type=jax.ShapeDtypeStruct((3 * NC * NPAD, D), jnp.float32),
    mesh=_mesh,
    scratch_types=[
        pltpu.VMEM_SHARED((NPAD, D), jnp.float32),
        pltpu.VMEM((64, D), jnp.float32),
        pltpu.VMEM((G, D // 2), jnp.int32),
        pltpu.VMEM((G, D // 2), jnp.int32),
        pltpu.VMEM((G, D), jnp.float32),
        pltpu.VMEM((G, D), jnp.float32),
        pltpu.VMEM((2 * G,), jnp.int32),
        pltpu.VMEM((2 * G,), jnp.int32),
        pltpu.VMEM((2 * G,), jnp.int32),
        pltpu.VMEM((2 * G,), jnp.int32),
        pltpu.SemaphoreType.DMA,
        pltpu.SemaphoreType.DMA,
        pltpu.SemaphoreType.DMA,
        pltpu.SemaphoreType.DMA,
        pltpu.SemaphoreType.DMA,
        pltpu.SemaphoreType.DMA,
        pltpu.SemaphoreType.DMA,
        pltpu.SemaphoreType.DMA,
    ],
    compiler_params=pltpu.CompilerParams(use_tc_tiling_on_sc=False,
                                         needs_layout_passes=False),
)
def _sc_scatter(table_hbm, comb_hbm, zrows_hbm, out_hbm,
                acc, zdbuf, gi0, gi1, g0, g1, xa0, xa1, xb0, xb1,
                semg0, semg1, sems0, sems1, semx0, semx1, semx2, semx3):
    c = lax.axis_index("c")
    s = lax.axis_index("s")
    nzc = RPT // 64
    coff = jnp.where(c == 0, 0, CH0)
    nch = jnp.where(c == 0, CH0, CH1)
    nquad = nch // 4
    himask = jnp.int32(-65536)

    def src(x):
        return x.at[pl.ds(0, G)]

    def dst(x):
        return x.at[pl.ds(G, G)]

    def unpack(gi, gf):
        def row(r, _):
            for k in range(4):
                v = gi[r, pl.ds(16 * k, 16)]
                gf[r, pl.ds(32 * k, 16)] = plsc.bitcast(
                    lax.shift_left(v, 16), jnp.float32)
                gf[r, pl.ds(32 * k + 16, 16)] = plsc.bitcast(
                    lax.bitwise_and(v, himask), jnp.float32)
            return _

        lax.fori_loop(0, G, row, None)

    for b in range(3):
        pltpu.sync_copy(zrows_hbm, zdbuf)

        def zero(h, _):
            pltpu.sync_copy(zdbuf, acc.at[pl.ds(s * RPT + h * 64, 64)])
            return _

        lax.fori_loop(0, nzc, zero, None)
        plsc.subcore_barrier()
        base = (b * NS + s) * CHT + coff
        pltpu.sync_copy(comb_hbm.at[base], xa0)
        pltpu.sync_copy(comb_hbm.at[base + 1], xa1)
        pltpu.async_copy(table_hbm.at[src(xa0)], gi0, semg0)
        pltpu.async_copy(table_hbm.at[src(xa1)], gi1, semg1)
        pltpu.async_copy(comb_hbm.at[base + 2], xb0, semx2)
        pltpu.async_copy(comb_hbm.at[base + 3], xb1, semx3)

        def quad(i, _):
            q = 4 * i
            pltpu.make_async_copy(table_hbm.at[src(xa0)], gi0, semg0).wait()
            unpack(gi0, g0)
            pltpu.async_copy(g0, acc.at[dst(xa0)], sems0, add=True)
            pltpu.make_async_copy(comb_hbm.at[base + q + 2], xb0, semx2).wait()
            pltpu.async_copy(table_hbm.at[src(xb0)], gi0, semg0)
            pltpu.make_async_copy(table_hbm.at[src(xa1)], gi1, semg1).wait()
            unpack(gi1, g1)
            pltpu.async_copy(g1, acc.at[dst(xa1)], sems1, add=True)
            pltpu.make_async_copy(comb_hbm.at[base + q + 3], xb1, semx3).wait()
            pltpu.async_copy(table_hbm.at[src(xb1)], gi1, semg1)
            pltpu.make_async_copy(g0, acc.at[dst(xa0)], sems0).wait()

            @pl.when(q + 4 < nch)
            def _():
                pltpu.async_copy(comb_hbm.at[base + q + 4], xa0, semx0)

            pltpu.make_async_copy(g1, acc.at[dst(xa1)], sems1).wait()

            @pl.when(q + 5 < nch)
            def _():
                pltpu.async_copy(comb_hbm.at[base + q + 5], xa1, semx1)

            pltpu.make_async_copy(table_hbm.at[src(xb0)], gi0, semg0).wait()
            unpack(gi0, g0)
            pltpu.async_copy(g0, acc.at[dst(xb0)], sems0, add=True)

            @pl.when(q + 4 < nch)
            def _():
                pltpu.make_async_copy(comb_hbm.at[base + q + 4], xa0,
                                      semx0).wait()
                pltpu.async_copy(table_hbm.at[src(xa0)], gi0, semg0)

            pltpu.make_async_copy(table_hbm.at[src(xb1)], gi1, semg1).wait()
            unpack(gi1, g1)
            pltpu.async_copy(g1, acc.at[dst(xb1)], sems1, add=True)

            @pl.when(q + 5 < nch)
            def _():
                pltpu.make_async_copy(comb_hbm.at[base + q + 5], xa1,
                                      semx1).wait()
                pltpu.async_copy(table_hbm.at[src(xa1)], gi1, semg1)

            pltpu.make_async_copy(g0, acc.at[dst(xb0)], sems0).wait()

            @pl.when(q + 6 < nch)
            def _():
                pltpu.async_copy(comb_hbm.at[base + q + 6], xb0, semx2)

            pltpu.make_async_copy(g1, acc.at[dst(xb1)], sems1).wait()

            @pl.when(q + 7 < nch)
            def _():
                pltpu.async_copy(comb_hbm.at[base + q + 7], xb1, semx3)

            return _

        lax.fori_loop(0, nquad, quad, None)
        plsc.subcore_barrier()
        off = (b * NC + c) * NPAD + s * RPT

        def dump(h, _):
            pltpu.sync_copy(acc.at[pl.ds(s * RPT + h * 64, 64)], zdbuf)
            pltpu.sync_copy(zdbuf, out_hbm.at[pl.ds(off + h * 64, 64)])
            return _

        lax.fori_loop(0, nzc, dump, None)
        plsc.subcore_barrier()


def _t0_body(x_ref, w_ref, degp_ref, scaled_ref, dinv_ref):
    deg = 1.0 + degp_ref[0, 0] + degp_ref[0, 1]
    dv = lax.rsqrt(deg)
    dinv_ref[0] = dv
    scaled_ref[0] = jnp.dot(x_ref[0], w_ref[0],
                            preferred_element_type=jnp.float32) * dv


def _t0(x, w0, degp):
    return pl.pallas_call(
        _t0_body,
        grid=(3, NB),
        in_specs=[
            pl.BlockSpec((1, BR, D), lambda b, i: (b, i, 0)),
            pl.BlockSpec((1, D, D), lambda b, i: (b, 0, 0)),
            pl.BlockSpec((1, 2, BR, 1), lambda b, i: (b, 0, i, 0)),
        ],
        out_specs=[
            pl.BlockSpec((1, BR, D), lambda b, i: (b, i, 0)),
            pl.BlockSpec((1, BR, 1), lambda b, i: (b, i, 0)),
        ],
        out_shape=[
            jax.ShapeDtypeStruct((3, N, D), jnp.float32),
            jax.ShapeDtypeStruct((3, N, 1), jnp.float32),
        ],
    )(x, w0, degp)


def _post_layer(x, sc, p0, p1, dv, bl, gl, bel):
    pre = (p0 + p1 + sc) * dv + bl[None, :]
    mu = jnp.mean(pre, axis=-1, keepdims=True)
    var = jnp.mean((pre - mu) ** 2, axis=-1, keepdims=True)
    h = (pre - mu) * lax.rsqrt(var + 1e-5) * gl[None, :] + bel[None, :]
    return x + jnp.maximum(h, 0.0)


def _tmid_body(x_ref, s_ref, p_ref, dinv_ref, b_ref, g_ref, be_ref, wn_ref,
               xn_ref, sn_ref):
    dv = dinv_ref[0]
    xn = _post_layer(x_ref[0], s_ref[0], p_ref[0, 0], p_ref[0, 1], dv,
                     b_ref[0, 0], g_ref[0, 0], be_ref[0, 0])
    xn_ref[0] = xn
    sn_ref[0] = jnp.dot(xn, wn_ref[0], preferred_element_type=jnp.float32) * dv


def _tmid(x, scaled, p, dinv, bl, gl, bel, wn):
    return pl.pallas_call(
        _tmid_body,
        grid=(3, NB),
        in_specs=[
            pl.BlockSpec((1, BR, D), lambda b, i: (b, i, 0)),
            pl.BlockSpec((1, BR, D), lambda b, i: (b, i, 0)),
            pl.BlockSpec((1, 2, BR, D), lambda b, i: (b, 0, i, 0)),
            pl.BlockSpec((1, BR, 1), lambda b, i: (b, i, 0)),
            pl.BlockSpec((1, 1, D), lambda b, i: (b, 0, 0)),
            pl.BlockSpec((1, 1, D), lambda b, i: (b, 0, 0)),
            pl.BlockSpec((1, 1, D), lambda b, i: (b, 0, 0)),
            pl.BlockSpec((1, D, D), lambda b, i: (b, 0, 0)),
        ],
        out_specs=[
            pl.BlockSpec((1, BR, D), lambda b, i: (b, i, 0)),
            pl.BlockSpec((1, BR, D), lambda b, i: (b, i, 0)),
        ],
        out_shape=[
            jax.ShapeDtypeStruct((3, N, D), jnp.float32),
            jax.ShapeDtypeStruct((3, N, D), jnp.float32),
        ],
    )(x, scaled, p, dinv, bl, gl, bel, wn)


def _tfin_body(x_ref, s_ref, p_ref, dinv_ref, b_ref, g_ref, be_ref, cw_ref,
               cb_ref, out_ref):
    b = pl.program_id(1)
    dv = dinv_ref[0]
    xn = _post_layer(x_ref[0], s_ref[0], p_ref[0, 0], p_ref[0, 1], dv,
                     b_ref[0, 0], g_ref[0, 0], be_ref[0, 0])
    contrib = jnp.dot(xn, cw_ref[0], preferred_element_type=jnp.float32)

    @pl.when(b == 0)
    def _():
        out_ref[...] = contrib + cb_ref[...]

    @pl.when(b > 0)
    def _():
        out_ref[...] += contrib


def _tfin(x, scaled, p, dinv, bl, gl, bel, cw, cb):
    return pl.pallas_call(
        _tfin_body,
        grid=(NB, 3),
        in_specs=[
            pl.BlockSpec((1, BR, D), lambda i, b: (b, i, 0)),
            pl.BlockSpec((1, BR, D), lambda i, b: (b, i, 0)),
            pl.BlockSpec((1, 2, BR, D), lambda i, b: (b, 0, i, 0)),
            pl.BlockSpec((1, BR, 1), lambda i, b: (b, i, 0)),
            pl.BlockSpec((1, 1, D), lambda i, b: (b, 0, 0)),
            pl.BlockSpec((1, 1, D), lambda i, b: (b, 0, 0)),
            pl.BlockSpec((1, 1, D), lambda i, b: (b, 0, 0)),
            pl.BlockSpec((1, D, C), lambda i, b: (b, 0, 0)),
            pl.BlockSpec((1, C), lambda i, b: (0, 0)),
        ],
        out_specs=pl.BlockSpec((BR, C), lambda i, b: (i, 0)),
        out_shape=jax.ShapeDtypeStruct((N, C), jnp.float32),
    )(x, scaled, p, dinv, bl, gl, bel, cw, cb)


def kernel(x_renormalized, edge_index_renormalized, x_vanilla,
           edge_index_vanilla, x_third, edge_index_third,
           W_ren, b_ren, g_ren, be_ren, W_van, b_van, g_van, be_van,
           W_thd, b_thd, g_thd, be_thd, clf_W, clf_b):
    x = jnp.stack([x_renormalized, x_vanilla, x_third])
    wm = jnp.stack([W_ren, W_van, W_thd])
    bm = jnp.stack([b_ren, b_van, b_thd])
    gm = jnp.stack([g_ren, g_van, g_thd])
    bem = jnp.stack([be_ren, be_van, be_thd])

    srcs = jnp.stack([edge_index_renormalized[0], edge_index_vanilla[0],
                      edge_index_third[0]]).astype(jnp.int32)
    dsts = jnp.stack([edge_index_renormalized[1], edge_index_vanilla[1],
                      edge_index_third[1]]).astype(jnp.int32)
    offs = (jnp.arange(3, dtype=jnp.int32) * N)[:, None]
    pad = EP - E
    src_p = jnp.concatenate(
        [srcs + offs, jnp.broadcast_to(offs, (3, pad))], axis=1)
    dst_p = jnp.concatenate(
        [dsts, jnp.full((3, pad), DUMMY, jnp.int32)], axis=1)
    src_hbm = src_p.reshape(3 * NS * CHT, G)
    dst_hbm = dst_p.reshape(3 * NS * CHT, G)
    comb_hbm = jnp.concatenate([src_hbm, dst_hbm], axis=1)

    zeros1 = jnp.zeros((RPT,), jnp.float32)
    ones_g = jnp.ones((G,), jnp.float32)
    zrows = jnp.zeros((64, D), jnp.float32)

    degp = _sc_degree(dst_hbm, zeros1, ones_g).reshape(3, NC, NPAD, 1)
    scaled, dinv = _t0(x, wm[:, 0], degp)

    for l in range(L):
        perm = scaled.reshape(3 * N, 4, 2, 16).swapaxes(2, 3)
        table = jax.lax.bitcast_convert_type(
            perm.astype(jnp.bfloat16).reshape(3 * N, D // 2, 2), jnp.int32)
        p = _sc_scatter(table, comb_hbm, zrows).reshape(3, NC, NPAD, D)
        if l < L - 1:
            x, scaled = _tmid(x, scaled, p, dinv, bm[:, l:l + 1],
                              gm[:, l:l + 1], bem[:, l:l + 1], wm[:, l + 1])
        else:
            out = _tfin(x, scaled, p, dinv, bm[:, l:l + 1], gm[:, l:l + 1],
                        bem[:, l:l + 1], clf_W.reshape(3, D, C),
                        clf_b.reshape(1, C))
    return out


# speedup vs baseline: 1.2845x; 1.0129x over previous
import functools

import jax
import jax.numpy as jnp
from jax import lax
from jax.experimental import pallas as pl
from jax.experimental.pallas import tpu as pltpu
from jax.experimental.pallas import tpu_sc as plsc

N = 10000
E = 320000
D = 128
L = 3
C = 10

NC = 2
NS = 16
NW = NC * NS
G = 80
CHT = 256
CH0 = 144
CH1 = CHT - CH0
EP = NS * CHT * G
NPAD = 10240
RPT = NPAD // NS
DUMMY = N
BR = 1000
NB = N // BR

_mesh = plsc.VectorSubcoreMesh(core_axis_name="c", subcore_axis_name="s")


@functools.partial(
    pl.kernel,
    out_type=jax.ShapeDtypeStruct((3 * NC * NPAD,), jnp.float32),
    mesh=_mesh,
    scratch_types=[
        pltpu.VMEM_SHARED((NPAD,), jnp.float32),
        pltpu.VMEM((G,), jnp.float32),
        pltpu.VMEM((CHT // 2, G), jnp.int32),
        pltpu.VMEM((RPT,), jnp.float32),
    ],
    compiler_params=pltpu.CompilerParams(use_tc_tiling_on_sc=False),
)
def _sc_degree(dst_hbm, zeros1_hbm, ones_hbm, out_hbm, deg_acc, ones_v, didx,
               zb):
    c = lax.axis_index("c")
    s = lax.axis_index("s")
    pltpu.sync_copy(ones_hbm, ones_v)
    pltpu.sync_copy(zeros1_hbm, zb)
    for b in range(3):
        pltpu.sync_copy(zb, deg_acc.at[pl.ds(s * RPT, RPT)])
        plsc.subcore_barrier()
        base = (b * NS + s) * CHT + c * (CHT // 2)
        pltpu.sync_copy(dst_hbm.at[pl.ds(base, CHT // 2)], didx)

        def chunk(j, _):
            pltpu.sync_copy(ones_v, deg_acc.at[didx.at[j]], add=True)
            return _

        lax.fori_loop(0, CHT // 2, chunk, None)
        plsc.subcore_barrier()
        off = (b * NC + c) * NPAD + s * RPT
        pltpu.sync_copy(deg_acc.at[pl.ds(s * RPT, RPT)], zb)
        pltpu.sync_copy(zb, out_hbm.at[pl.ds(off, RPT)])
        pltpu.sync_copy(zeros1_hbm, zb)
        plsc.subcore_barrier()


@functools.partial(
    pl.kernel,
    out_type=jax.ShapeDtypeStruct((3 * NC * NPAD, D), jnp.float32),
    mesh=_mesh,
    scratch_types=[
        pltpu.VMEM_SHARED((NPAD, D), jnp.float32),
        pltpu.VMEM((64, D), jnp.float32),
        pltpu.VMEM((G, D // 2), jnp.int32),
        pltpu.VMEM((G, D // 2), jnp.int32),
        pltpu.VMEM((G, D), jnp.float32),
        pltpu.VMEM((G, D), jnp.float32),
        pltpu.VMEM((2 * G,), jnp.int32),
        pltpu.VMEM((2 * G,), jnp.int32),
        pltpu.VMEM((2 * G,), jnp.int32),
        pltpu.VMEM((2 * G,), jnp.int32),
        pltpu.SemaphoreType.DMA,
        pltpu.SemaphoreType.DMA,
        pltpu.SemaphoreType.DMA,
        pltpu.SemaphoreType.DMA,
        pltpu.SemaphoreType.DMA,
        pltpu.SemaphoreType.DMA,
        pltpu.SemaphoreType.DMA,
        pltpu.SemaphoreType.DMA,
    ],
    compiler_params=pltpu.CompilerParams(use_tc_tiling_on_sc=False,
                                         needs_layout_passes=False),
)
def _sc_scatter(table_hbm, comb_hbm, zrows_hbm, out_hbm,
                acc, zdbuf, gi0, gi1, g0, g1, xa0, xa1, xb0, xb1,
                semg0, semg1, sems0, sems1, semx0, semx1, semx2, semx3):
    c = lax.axis_index("c")
    s = lax.axis_index("s")
    nzc = RPT // 64
    coff = jnp.where(c == 0, 0, CH0)
    nch = jnp.where(c == 0, CH0, CH1)
    nquad = nch // 4
    himask = jnp.int32(-65536)

    def src(x):
        return x.at[pl.ds(0, G)]

    def dst(x):
        return x.at[pl.ds(G, G)]

    def unpack(gi, gf):
        def row(r, _):
            for k in range(4):
                v = gi[r, pl.ds(16 * k, 16)]
                gf[r, pl.ds(32 * k, 16)] = plsc.bitcast(
                    lax.shift_left(v, 16), jnp.float32)
                gf[r, pl.ds(32 * k + 16, 16)] = plsc.bitcast(
                    lax.bitwise_and(v, himask), jnp.float32)
            return _

        lax.fori_loop(0, G, row, None)

    for b in range(3):
        pltpu.sync_copy(zrows_hbm, zdbuf)

        def zero(h, _):
            pltpu.sync_copy(zdbuf, acc.at[pl.ds(s * RPT + h * 64, 64)])
            return _

        lax.fori_loop(0, nzc, zero, None)
        plsc.subcore_barrier()
        base = (b * NS + s) * CHT + coff
        pltpu.sync_copy(comb_hbm.at[base], xa0)
        pltpu.sync_copy(comb_hbm.at[base + 1], xa1)
        pltpu.async_copy(table_hbm.at[src(xa0)], gi0, semg0)
        pltpu.async_copy(table_hbm.at[src(xa1)], gi1, semg1)
        pltpu.async_copy(comb_hbm.at[base + 2], xb0, semx2)
        pltpu.async_copy(comb_hbm.at[base + 3], xb1, semx3)

        def quad(i, _):
            q = 4 * i
            pltpu.make_async_copy(table_hbm.at[src(xa0)], gi0, semg0).wait()
            unpack(gi0, g0)
            pltpu.async_copy(g0, acc.at[dst(xa0)], sems0, add=True)
            pltpu.make_async_copy(comb_hbm.at[base + q + 2], xb0, semx2).wait()
            pltpu.async_copy(table_hbm.at[src(xb0)], gi0, semg0)
            pltpu.make_async_copy(table_hbm.at[src(xa1)], gi1, semg1).wait()
            unpack(gi1, g1)
            pltpu.async_copy(g1, acc.at[dst(xa1)], sems1, add=True)
            pltpu.make_async_copy(comb_hbm.at[base + q + 3], xb1, semx3).wait()
            pltpu.async_copy(table_hbm.at[src(xb1)], gi1, semg1)
            pltpu.make_async_copy(g0, acc.at[dst(xa0)], sems0).wait()

            @pl.when(q + 4 < nch)
            def _():
                pltpu.async_copy(comb_hbm.at[base + q + 4], xa0, semx0)

            pltpu.make_async_copy(g1, acc.at[dst(xa1)], sems1).wait()

            @pl.when(q + 5 < nch)
            def _():
                pltpu.async_copy(comb_hbm.at[base + q + 5], xa1, semx1)

            pltpu.make_async_copy(table_hbm.at[src(xb0)], gi0, semg0).wait()
            unpack(gi0, g0)
            pltpu.async_copy(g0, acc.at[dst(xb0)], sems0, add=True)

            @pl.when(q + 4 < nch)
            def _():
                pltpu.make_async_copy(comb_hbm.at[base + q + 4], xa0,
                                      semx0).wait()
                pltpu.async_copy(table_hbm.at[src(xa0)], gi0, semg0)

            pltpu.make_async_copy(table_hbm.at[src(xb1)], gi1, semg1).wait()
            unpack(gi1, g1)
            pltpu.async_copy(g1, acc.at[dst(xb1)], sems1, add=True)

            @pl.when(q + 5 < nch)
            def _():
                pltpu.make_async_copy(comb_hbm.at[base + q + 5], xa1,
                                      semx1).wait()
                pltpu.async_copy(table_hbm.at[src(xa1)], gi1, semg1)

            pltpu.make_async_copy(g0, acc.at[dst(xb0)], sems0).wait()

            @pl.when(q + 6 < nch)
            def _():
                pltpu.async_copy(comb_hbm.at[base + q + 6], xb0, semx2)

            pltpu.make_async_copy(g1, acc.at[dst(xb1)], sems1).wait()

            @pl.when(q + 7 < nch)
            def _():
                pltpu.async_copy(comb_hbm.at[base + q + 7], xb1, semx3)

            return _

        lax.fori_loop(0, nquad, quad, None)
        plsc.subcore_barrier()
        off = (b * NC + c) * NPAD + s * RPT

        def dump(h, _):
            pltpu.sync_copy(acc.at[pl.ds(s * RPT + h * 64, 64)], zdbuf)
            pltpu.sync_copy(zdbuf, out_hbm.at[pl.ds(off + h * 64, 64)])
            return _

        lax.fori_loop(0, nzc, dump, None)
        plsc.subcore_barrier()


def _t0_body(x_ref, w_ref, degp_ref, scaled_ref, dinv_ref):
    deg = 1.0 + degp_ref[0, 0] + degp_ref[0, 1]
    dv = lax.rsqrt(deg)
    dinv_ref[0] = dv
    scaled_ref[0] = jnp.dot(x_ref[0], w_ref[0],
                            preferred_element_type=jnp.float32) * dv


def _t0(x, w0, degp):
    return pl.pallas_call(
        _t0_body,
        grid=(3, NB),
        in_specs=[
            pl.BlockSpec((1, BR, D), lambda b, i: (b, i, 0)),
            pl.BlockSpec((1, D, D), lambda b, i: (b, 0, 0)),
            pl.BlockSpec((1, 2, BR, 1), lambda b, i: (b, 0, i, 0)),
        ],
        out_specs=[
            pl.BlockSpec((1, BR, D), lambda b, i: (b, i, 0)),
            pl.BlockSpec((1, BR, 1), lambda b, i: (b, i, 0)),
        ],
        out_shape=[
            jax.ShapeDtypeStruct((3, N, D), jnp.float32),
            jax.ShapeDtypeStruct((3, N, 1), jnp.float32),
        ],
    )(x, w0, degp)


def _post_layer(x, sc, p0, p1, dv, bl, gl, bel):
    pre = (p0 + p1 + sc) * dv + bl[None, :]
    mu = jnp.mean(pre, axis=-1, keepdims=True)
    var = jnp.mean((pre - mu) ** 2, axis=-1, keepdims=True)
    h = (pre - mu) * lax.rsqrt(var + 1e-5) * gl[None, :] + bel[None, :]
    return x + jnp.maximum(h, 0.0)


def _tmid_body(x_ref, s_ref, p_ref, dinv_ref, b_ref, g_ref, be_ref, wn_ref,
               xn_ref, sn_ref):
    dv = dinv_ref[0]
    xn = _post_layer(x_ref[0], s_ref[0], p_ref[0, 0], p_ref[0, 1], dv,
                     b_ref[0, 0], g_ref[0, 0], be_ref[0, 0])
    xn_ref[0] = xn
    sn_ref[0] = jnp.dot(xn, wn_ref[0], preferred_element_type=jnp.float32) * dv


def _tmid(x, scaled, p, dinv, bl, gl, bel, wn):
    return pl.pallas_call(
        _tmid_body,
        grid=(3, NB),
        in_specs=[
            pl.BlockSpec((1, BR, D), lambda b, i: (b, i, 0)),
            pl.BlockSpec((1, BR, D), lambda b, i: (b, i, 0)),
            pl.BlockSpec((1, 2, BR, D), lambda b, i: (b, 0, i, 0)),
            pl.BlockSpec((1, BR, 1), lambda b, i: (b, i, 0)),
            pl.BlockSpec((1, 1, D), lambda b, i: (b, 0, 0)),
            pl.BlockSpec((1, 1, D), lambda b, i: (b, 0, 0)),
            pl.BlockSpec((1, 1, D), lambda b, i: (b, 0, 0)),
            pl.BlockSpec((1, D, D), lambda b, i: (b, 0, 0)),
        ],
        out_specs=[
            pl.BlockSpec((1, BR, D), lambda b, i: (b, i, 0)),
            pl.BlockSpec((1, BR, D), lambda b, i: (b, i, 0)),
        ],
        out_shape=[
            jax.ShapeDtypeStruct((3, N, D), jnp.float32),
            jax.ShapeDtypeStruct((3, N, D), jnp.float32),
        ],
    )(x, scaled, p, dinv, bl, gl, bel, wn)


def _tfin_body(x_ref, s_ref, p_ref, dinv_ref, b_ref, g_ref, be_ref, cw_ref,
               cb_ref, out_ref):
    b = pl.program_id(1)
    dv = dinv_ref[0]
    xn = _post_layer(x_ref[0], s_ref[0], p_ref[0, 0], p_ref[0, 1], dv,
                     b_ref[0, 0], g_ref[0, 0], be_ref[0, 0])
    contrib = jnp.dot(xn, cw_ref[0], preferred_element_type=jnp.float32)

    @pl.when(b == 0)
    def _():
        out_ref[...] = contrib + cb_ref[...]

    @pl.when(b > 0)
    def _():
        out_ref[...] += contrib


def _tfin(x, scaled, p, dinv, bl, gl, bel, cw, cb):
    return pl.pallas_call(
        _tfin_body,
        grid=(NB, 3),
        in_specs=[
            pl.BlockSpec((1, BR, D), lambda i, b: (b, i, 0)),
            pl.BlockSpec((1, BR, D), lambda i, b: (b, i, 0)),
            pl.BlockSpec((1, 2, BR, D), lambda i, b: (b, 0, i, 0)),
            pl.BlockSpec((1, BR, 1), lambda i, b: (b, i, 0)),
            pl.BlockSpec((1, 1, D), lambda i, b: (b, 0, 0)),
            pl.BlockSpec((1, 1, D), lambda i, b: (b, 0, 0)),
            pl.BlockSpec((1, 1, D), lambda i, b: (b, 0, 0)),
            pl.BlockSpec((1, D, C), lambda i, b: (b, 0, 0)),
            pl.BlockSpec((1, C), lambda i, b: (0, 0)),
        ],
        out_specs=pl.BlockSpec((BR, C), lambda i, b: (i, 0)),
        out_shape=jax.ShapeDtypeStruct((N, C), jnp.float32),
    )(x, scaled, p, dinv, bl, gl, bel, cw, cb)


def kernel(x_renormalized, edge_index_renormalized, x_vanilla,
           edge_index_vanilla, x_third, edge_index_third,
           W_ren, b_ren, g_ren, be_ren, W_van, b_van, g_van, be_van,
           W_thd, b_thd, g_thd, be_thd, clf_W, clf_b):
    x = jnp.stack([x_renormalized, x_vanilla, x_third])
    wm = jnp.stack([W_ren, W_van, W_thd])
    bm = jnp.stack([b_ren, b_van, b_thd])
    gm = jnp.stack([g_ren, g_van, g_thd])
    bem = jnp.stack([be_ren, be_van, be_thd])

    srcs = jnp.stack([edge_index_renormalized[0], edge_index_vanilla[0],
                      edge_index_third[0]]).astype(jnp.int32)
    dsts = jnp.stack([edge_index_renormalized[1], edge_index_vanilla[1],
                      edge_index_third[1]]).astype(jnp.int32)
    offs = (jnp.arange(3, dtype=jnp.int32) * N)[:, None]
    pad = EP - E
    src_p = jnp.concatenate(
        [srcs + offs, jnp.broadcast_to(offs, (3, pad))], axis=1)
    dst_p = jnp.concatenate(
        [dsts, jnp.full((3, pad), DUMMY, jnp.int32)], axis=1)
    src_hbm = src_p.reshape(3 * NS * CHT, G)
    dst_hbm = dst_p.reshape(3 * NS * CHT, G)
    comb_hbm = jnp.concatenate([src_hbm, dst_hbm], axis=1)

    zeros1 = jnp.zeros((RPT,), jnp.float32)
    ones_g = jnp.ones((G,), jnp.float32)
    zrows = jnp.zeros((64, D), jnp.float32)

    degp = _sc_degree(dst_hbm, zeros1, ones_g).reshape(3, NC, NPAD, 1)
    scaled, dinv = _t0(x, wm[:, 0], degp)

    for l in range(L):
        perm = scaled.reshape(3 * N, 4, 2, 16).swapaxes(2, 3)
        table = jax.lax.bitcast_convert_type(
            perm.astype(jnp.bfloat16).reshape(3 * N, D // 2, 2), jnp.int32)
        p = _sc_scatter(table, comb_hbm, zrows).reshape(3, NC, NPAD, D)
        if l < L - 1:
            x, scaled = _tmid(x, scaled, p, dinv, bm[:, l:l + 1],
                              gm[:, l:l + 1], bem[:, l:l + 1], wm[:, l + 1])
        else:
            out = _tfin(x, scaled, p, dinv, bm[:, l:l + 1], gm[:, l:l + 1],
                        bem[:, l:l + 1], clf_W.reshape(3, D, C),
                        clf_b.reshape(1, C))
    return out
